# baseline (speedup 1.0000x reference)
import jax
import jax.numpy as jnp
from jax.experimental import pallas as pl
from jax.experimental.pallas import tpu as pltpu

_NUM_EXPERTS = 16
_BLOCK_M = 1024

_DNT = (((1,), (1,)), ((), ()))


def _fused_gating_kernel(x_ref, w1_ref, b1_ref, w2_ref, b2_ref, w3_ref, b3_ref,
                         out_ref):
    x = x_ref[...]
    h = jnp.maximum(jax.lax.dot_general(x, w1_ref[...], _DNT) + b1_ref[...],
                    0.0)
    h = jnp.maximum(jax.lax.dot_general(h, w2_ref[...], _DNT) + b2_ref[...],
                    0.0)
    logits = jax.lax.dot_general(h, w3_ref[...], _DNT) + b3_ref[...]
    m = jnp.max(logits, axis=1, keepdims=True)
    col = jax.lax.broadcasted_iota(jnp.int32, logits.shape, 1)
    idx = jnp.min(jnp.where(logits == m, col, _NUM_EXPERTS), axis=1,
                  keepdims=True)
    out_ref[...] = (col == idx).astype(jnp.float32)


def kernel(features, W1, b1, W2, b2, W3, b3):
    n_tokens, input_size = features.shape
    hidden = W1.shape[0]
    hidden2 = W2.shape[0]
    n_experts = W3.shape[0]

    b1r = b1.reshape(1, hidden)
    b2r = b2.reshape(1, hidden2)
    b3r = b3.reshape(1, n_experts)

    bm = min(_BLOCK_M, n_tokens)
    grid = (n_tokens // bm,)

    return pl.pallas_call(
        _fused_gating_kernel,
        grid=grid,
        in_specs=[
            pl.BlockSpec((bm, input_size), lambda i: (i, 0)),
            pl.BlockSpec((hidden, input_size), lambda i: (0, 0)),
            pl.BlockSpec((1, hidden), lambda i: (0, 0)),
            pl.BlockSpec((hidden2, hidden), lambda i: (0, 0)),
            pl.BlockSpec((1, hidden2), lambda i: (0, 0)),
            pl.BlockSpec((n_experts, hidden2), lambda i: (0, 0)),
            pl.BlockSpec((1, n_experts), lambda i: (0, 0)),
        ],
        out_specs=pl.BlockSpec((bm, n_experts), lambda i: (i, 0)),
        out_shape=jax.ShapeDtypeStruct((n_tokens, n_experts), jnp.float32),
        compiler_params=pltpu.CompilerParams(
            dimension_semantics=("parallel",),
        ),
    )(features, W1, b1r, W2, b2r, W3, b3r)


# speedup vs baseline: 1.0887x; 1.0887x over previous
import jax
import jax.numpy as jnp
from jax.experimental import pallas as pl
from jax.experimental.pallas import tpu as pltpu

_NUM_EXPERTS = 16
_BLOCK_M = 4096

_DNT = (((1,), (1,)), ((), ()))


def _fused_gating_kernel(x_ref, w1_ref, b1_ref, w2_ref, b2_ref, w3_ref, b3_ref,
                         out_ref):
    x = x_ref[...]
    h = jnp.maximum(jax.lax.dot_general(x, w1_ref[...], _DNT) + b1_ref[...],
                    0.0)
    h = jnp.maximum(jax.lax.dot_general(h, w2_ref[...], _DNT) + b2_ref[...],
                    0.0)
    logits = jax.lax.dot_general(h, w3_ref[...], _DNT) + b3_ref[...]
    m = jnp.max(logits, axis=1, keepdims=True)
    col = jax.lax.broadcasted_iota(jnp.int32, logits.shape, 1)
    idx = jnp.min(jnp.where(logits == m, col, _NUM_EXPERTS), axis=1,
                  keepdims=True)
    out_ref[...] = (col == idx).astype(jnp.float32)


def kernel(features, W1, b1, W2, b2, W3, b3):
    n_tokens, input_size = features.shape
    hidden = W1.shape[0]
    hidden2 = W2.shape[0]
    n_experts = W3.shape[0]

    b1r = b1.reshape(1, hidden)
    b2r = b2.reshape(1, hidden2)
    b3r = b3.reshape(1, n_experts)

    bm = min(_BLOCK_M, n_tokens)
    grid = (n_tokens // bm,)

    return pl.pallas_call(
        _fused_gating_kernel,
        grid=grid,
        in_specs=[
            pl.BlockSpec((bm, input_size), lambda i: (i, 0)),
            pl.BlockSpec((hidden, input_size), lambda i: (0, 0)),
            pl.BlockSpec((1, hidden), lambda i: (0, 0)),
            pl.BlockSpec((hidden2, hidden), lambda i: (0, 0)),
            pl.BlockSpec((1, hidden2), lambda i: (0, 0)),
            pl.BlockSpec((n_experts, hidden2), lambda i: (0, 0)),
            pl.BlockSpec((1, n_experts), lambda i: (0, 0)),
        ],
        out_specs=pl.BlockSpec((bm, n_experts), lambda i: (i, 0)),
        out_shape=jax.ShapeDtypeStruct((n_tokens, n_experts), jnp.float32),
        compiler_params=pltpu.CompilerParams(
            dimension_semantics=("parallel",),
        ),
    )(features, W1, b1r, W2, b2r, W3, b3r)


# speedup vs baseline: 1.3223x; 1.2147x over previous
import jax
import jax.numpy as jnp
from jax.experimental import pallas as pl
from jax.experimental.pallas import tpu as pltpu

_NUM_EXPERTS = 16
_BLOCK_M = 2048

_DNT = (((1,), (1,)), ((), ()))
_DNN = (((1,), (0,)), ((), ()))


def _fused_gating_kernel(x_ref, w1_ref, b1_ref, w2_ref, b2_ref, w3_ref, b3_ref,
                         out_ref):
    x = x_ref[...]
    h = jnp.maximum(jax.lax.dot_general(w1_ref[...], x, _DNT) + b1_ref[...],
                    0.0)
    h = jnp.maximum(jax.lax.dot_general(w2_ref[...], h, _DNN) + b2_ref[...],
                    0.0)
    lt = jax.lax.dot_general(w3_ref[...], h, _DNN) + b3_ref[...]
    m = jnp.max(lt, axis=0, keepdims=True)
    row = jax.lax.broadcasted_iota(jnp.int32, lt.shape, 0)
    idx = jnp.min(jnp.where(lt == m, row, _NUM_EXPERTS), axis=0,
                  keepdims=True)
    out_ref[...] = (row == idx).astype(jnp.float32)


def kernel(features, W1, b1, W2, b2, W3, b3):
    n_tokens, input_size = features.shape
    hidden = W1.shape[0]
    hidden2 = W2.shape[0]
    n_experts = W3.shape[0]

    b1c = b1.reshape(hidden, 1)
    b2c = b2.reshape(hidden2, 1)
    b3c = b3.reshape(n_experts, 1)

    bm = min(_BLOCK_M, n_tokens)
    grid = (n_tokens // bm,)

    onehot_t = pl.pallas_call(
        _fused_gating_kernel,
        grid=grid,
        in_specs=[
            pl.BlockSpec((bm, input_size), lambda i: (i, 0)),
            pl.BlockSpec((hidden, input_size), lambda i: (0, 0)),
            pl.BlockSpec((hidden, 1), lambda i: (0, 0)),
            pl.BlockSpec((hidden2, hidden), lambda i: (0, 0)),
            pl.BlockSpec((hidden2, 1), lambda i: (0, 0)),
            pl.BlockSpec((n_experts, hidden2), lambda i: (0, 0)),
            pl.BlockSpec((n_experts, 1), lambda i: (0, 0)),
        ],
        out_specs=pl.BlockSpec((n_experts, bm), lambda i: (0, i)),
        out_shape=jax.ShapeDtypeStruct((n_experts, n_tokens), jnp.float32),
        compiler_params=pltpu.CompilerParams(
            dimension_semantics=("arbitrary",),
        ),
    )(features, W1, b1c, W2, b2c, W3, b3c)
    return onehot_t.T
